# baseline (speedup 1.0000x reference)
import functools

import jax
import jax.numpy as jnp
from jax import lax
from jax.experimental import pallas as pl
from jax.experimental.pallas import tpu as pltpu
from jax.experimental.pallas import tpu_sc as plsc

N, T = 4096, 200
D = 64
V = 1000000
B = N * T
NC, NS = 2, 16
NW = NC * NS
K = 128
NBLK = B // K
BLK_PER_W = NBLK // NW
NBUF = 4
SB_PER_W = BLK_PER_W // 2
NB_N = N // K


@functools.partial(
    pl.kernel,
    mesh=plsc.VectorSubcoreMesh(core_axis_name="c", subcore_axis_name="s"),
    out_type=jax.ShapeDtypeStruct((T, D, N), jnp.float32),
    compiler_params=pltpu.CompilerParams(
        use_tc_tiling_on_sc=True, needs_layout_passes=False),
    scratch_types=(
        [pltpu.VMEM((BLK_PER_W, K), jnp.int32)]
        + [pltpu.VMEM((K, 128), jnp.float32)] * NBUF
        + [pltpu.VMEM((D, 2 * K), jnp.float32)] * 2
        + [pltpu.SemaphoreType.DMA] * (NBUF + 2)
    ),
)
def _gather_kernel(table_hbm, idx_hbm, out_hbm, idx_v, *scratch):
    rows = scratch[:NBUF]
    tbuf = scratch[NBUF:NBUF + 2]
    gsem = scratch[NBUF + 2:2 * NBUF + 2]
    wsem = scratch[2 * NBUF + 2:]
    wid = lax.axis_index("s") * NC + lax.axis_index("c")
    pltpu.sync_copy(idx_hbm.at[pl.ds(wid * BLK_PER_W, BLK_PER_W)], idx_v)
    base = wid * BLK_PER_W

    lane = lax.iota(jnp.int32, 16)
    row_sel = [lane + 16 * k for k in range(K // 16)]

    def start_g2(sb, p):
        for h in range(2):
            b = 2 * p + h
            pltpu.async_copy(table_hbm.at[idx_v.at[2 * sb + h]], rows[b], gsem[b])

    def wait_g2(sb, p):
        for h in range(2):
            b = 2 * p + h
            pltpu.make_async_copy(
                table_hbm.at[idx_v.at[2 * sb + h]], rows[b], gsem[b]).wait()

    def _dst(sb):
        r = base + 2 * sb
        return out_hbm.at[r >> 5, :, pl.ds((r & 31) * K, 2 * K)]

    def start_wb(sb, p):
        pltpu.async_copy(tbuf[p], _dst(sb), wsem[p])

    def wait_wb(sb, p):
        pltpu.make_async_copy(tbuf[p], _dst(sb), wsem[p]).wait()

    def transpose2(p):
        @plsc.parallel_loop(0, D, unroll=4)
        def _per_d(d):
            col = lax.broadcast(d, (16,))
            for h in range(2):
                for k in range(K // 16):
                    vals = plsc.load_gather(rows[2 * p + h], [row_sel[k], col])
                    tbuf[p][d, pl.ds(128 * h + 16 * k, 16)] = vals

    for p in range(2):
        start_g2(p, p)

    for p in range(2):
        wait_g2(p, p)
        transpose2(p)
        start_wb(p, p)
        start_g2(p + 2, p)

    def body(i, carry):
        for p in range(2):
            sb = 2 * i + p
            wait_g2(sb, p)
            wait_wb(sb - 2, p)
            transpose2(p)
            start_wb(sb, p)
            start_g2(sb + 2, p)
        return carry

    lax.fori_loop(1, SB_PER_W // 2 - 1, body, 0)

    for p in range(2):
        sb = SB_PER_W - 2 + p
        wait_g2(sb, p)
        wait_wb(sb - 2, p)
        transpose2(p)
        start_wb(sb, p)
    for p in range(2):
        wait_wb(SB_PER_W - 2 + p, p)


def kernel(x, W_embed):
    idx = jnp.transpose(x).reshape(NBLK, K).astype(jnp.int32)
    Wp = jnp.pad(W_embed, ((0, 0), (0, 128 - D)))
    out = _gather_kernel(Wp, idx)
    return out.transpose(2, 0, 1)
